# baseline (speedup 1.0000x reference)
import functools

import jax
import jax.numpy as jnp
from jax import lax
from jax.experimental import pallas as pl
from jax.experimental.pallas import tpu as pltpu
from jax.experimental.pallas import tpu_sc as plsc

B = 16384
D = 64
NC = 2
NS = 16
NW = NC * NS
B_PER_W = B // NW
CHUNK = 128
NCHUNK = B_PER_W // CHUNK

_mesh = plsc.VectorSubcoreMesh(core_axis_name="c", subcore_axis_name="s")


@functools.partial(
    pl.kernel,
    mesh=_mesh,
    out_type=jax.ShapeDtypeStruct((NW, B_PER_W, D), jnp.float32),
    scratch_types=[
        pltpu.VMEM((NCHUNK, CHUNK), jnp.int32),
        pltpu.VMEM((B_PER_W, D), jnp.float32),
        pltpu.SemaphoreType.DMA,
    ],
    compiler_params=pltpu.CompilerParams(use_tc_tiling_on_sc=False),
)
def _embed(table_hbm, labels_hbm, out_hbm, idx_v, rows_v, sem):
    wid = lax.axis_index("s") * NC + lax.axis_index("c")
    pltpu.sync_copy(labels_hbm.at[wid], idx_v)
    copies = []
    for j in range(NCHUNK):
        copies.append(
            pltpu.async_copy(
                table_hbm.at[idx_v.at[j]],
                rows_v.at[pl.ds(j * CHUNK, CHUNK)],
                sem,
            )
        )
    for c in copies:
        c.wait()
    pltpu.sync_copy(rows_v, out_hbm.at[wid])


def kernel(labels, table):
    labels_3d = labels.astype(jnp.int32).reshape(NW, NCHUNK, CHUNK)
    out = _embed(table, labels_3d)
    return out.reshape(B, D)


# speedup vs baseline: 1.8644x; 1.8644x over previous
import functools

import jax
import jax.numpy as jnp
from jax import lax
from jax.experimental import pallas as pl
from jax.experimental.pallas import tpu as pltpu
from jax.experimental.pallas import tpu_sc as plsc

B = 16384
D = 64
V = 1000001
NC = 2
NS = 16
NW = NC * NS
B_PER_W = B // NW
L = 16
G = 128

_mesh = plsc.VectorSubcoreMesh(core_axis_name="c", subcore_axis_name="s")


@functools.partial(
    pl.kernel,
    mesh=_mesh,
    out_type=jax.ShapeDtypeStruct((8, 8, B), jnp.float32),
    scratch_types=[
        pltpu.VMEM((B_PER_W,), jnp.int32),
        pltpu.VMEM((2, 8, 8, G), jnp.float32),
        pltpu.VMEM((8, 8, B_PER_W), jnp.float32),
        pltpu.SemaphoreType.DMA,
        pltpu.SemaphoreType.DMA,
    ],
    compiler_params=pltpu.CompilerParams(
        use_tc_tiling_on_sc=True, needs_layout_passes=False
    ),
)
def _embed(tableT3_hbm, labels_hbm, outT3_hbm, lab_v, gran_v, blk_v, sem0,
           sem1):
    wid = lax.axis_index("s") * NC + lax.axis_index("c")
    pltpu.sync_copy(labels_hbm.at[wid], lab_v)

    sems = [sem0, sem1]
    lane_iota = lax.iota(jnp.int32, L)
    ones = jnp.ones((L,), jnp.int32)
    zeros = jnp.zeros((L,), jnp.int32)
    eight = jnp.full((L,), 8, jnp.int32)
    pair_i0 = lax.div(lane_iota, eight)
    pair_a = lax.rem(lane_iota, eight)

    def label_scalar(i):
        grp = lab_v[pl.ds((i // L) * L, L)]
        sel = jnp.where(lax.eq(lane_iota, lax.rem(i, L) * ones), grp, zeros)
        return lax.reduce_max(sel, axes=(0,))

    def fire(i, b):
        c = label_scalar(i)
        base = pl.multiple_of((c // G) * G, G)
        pltpu.async_copy(
            tableT3_hbm.at[:, :, pl.ds(base, G)], gran_v.at[b], sems[b]
        )

    def wait(b):
        pltpu.make_async_copy(
            tableT3_hbm.at[:, :, pl.ds(0, G)], gran_v.at[b], sems[b]
        ).wait()

    def extract(i, b):
        c = label_scalar(i)
        cw = lax.rem(c, G) * ones
        ivec = i * ones
        buf = gran_v.at[b]
        for g in range(4):
            i_idx = lax.add(pair_i0, jnp.full((L,), g * 2, jnp.int32))
            vals = plsc.load_gather(buf, [i_idx, pair_a, cw])
            plsc.store_scatter(blk_v, [i_idx, pair_a, ivec], vals)

    fire(0, 0)
    fire(1, 1)

    def pipe(k, _):
        i0 = k * 2
        wait(0)
        extract(i0, 0)
        fire(i0 + 2, 0)
        wait(1)
        extract(i0 + 1, 1)
        fire(i0 + 3, 1)
        return 0

    lax.fori_loop(0, (B_PER_W - 2) // 2, pipe, 0)
    for t in (B_PER_W - 2, B_PER_W - 1):
        wait(t % 2)
        extract(t, t % 2)

    pltpu.sync_copy(blk_v, outT3_hbm.at[:, :, pl.ds(wid * B_PER_W, B_PER_W)])


def kernel(labels, table):
    tableT3 = table.T.reshape(8, 8, V)
    labels_2d = labels.astype(jnp.int32).reshape(NW, B_PER_W)
    outT3 = _embed(tableT3, labels_2d)
    return outT3.reshape(D, B).T


# speedup vs baseline: 3.0036x; 1.6110x over previous
import functools

import jax
import jax.numpy as jnp
from jax import lax
from jax.experimental import pallas as pl
from jax.experimental.pallas import tpu as pltpu
from jax.experimental.pallas import tpu_sc as plsc

B = 16384
D = 64
V = 1000001
NC = 2
NS = 16
NW = NC * NS
B_PER_W = B // NW
L = 16
G = 128

_mesh = plsc.VectorSubcoreMesh(core_axis_name="c", subcore_axis_name="s")


@functools.partial(
    pl.kernel,
    mesh=_mesh,
    out_type=jax.ShapeDtypeStruct((8, 8, B), jnp.float32),
    scratch_types=[
        pltpu.VMEM((B_PER_W,), jnp.int32),
        pltpu.VMEM((8, 8, 8, G), jnp.float32),
        pltpu.VMEM((8, 8, B_PER_W), jnp.float32),
        pltpu.SemaphoreType.DMA,
        pltpu.SemaphoreType.DMA,
        pltpu.SemaphoreType.DMA,
        pltpu.SemaphoreType.DMA,
        pltpu.SemaphoreType.DMA,
        pltpu.SemaphoreType.DMA,
        pltpu.SemaphoreType.DMA,
        pltpu.SemaphoreType.DMA,
    ],
    compiler_params=pltpu.CompilerParams(
        use_tc_tiling_on_sc=True, needs_layout_passes=False
    ),
)
def _embed(tableT3_hbm, labels_hbm, outT3_hbm, lab_v, gran_v, blk_v, *sems):
    wid = lax.axis_index("s") * NC + lax.axis_index("c")
    pltpu.sync_copy(labels_hbm.at[wid], lab_v)
    lane_iota = lax.iota(jnp.int32, L)
    ones = jnp.ones((L,), jnp.int32)
    zeros = jnp.zeros((L,), jnp.int32)
    eight = jnp.full((L,), 8, jnp.int32)
    pair_i0 = lax.div(lane_iota, eight)
    pair_a = lax.rem(lane_iota, eight)

    def label_scalar(i):
        grp = lab_v[pl.ds((i // L) * L, L)]
        sel = jnp.where(lax.eq(lane_iota, lax.rem(i, L) * ones), grp, zeros)
        return lax.reduce_max(sel, axes=(0,))

    def fire(i, b):
        c = label_scalar(i)
        base = pl.multiple_of((c // G) * G, G)
        pltpu.async_copy(
            tableT3_hbm.at[:, :, pl.ds(base, G)], gran_v.at[b], sems[b]
        )

    def wait(b):
        pltpu.make_async_copy(
            tableT3_hbm.at[:, :, pl.ds(0, G)], gran_v.at[b], sems[b]
        ).wait()

    def extract(i, b):
        c = label_scalar(i)
        cw = lax.rem(c, G) * ones
        ivec = i * ones
        buf = gran_v.at[b]
        for g in range(4):
            i_idx = lax.add(pair_i0, jnp.full((L,), g * 2, jnp.int32))
            vals = plsc.load_gather(buf, [i_idx, pair_a, cw])
            plsc.store_scatter(blk_v, [i_idx, pair_a, ivec], vals)

    NBUF = 8
    for b in range(NBUF):
        fire(b, b)

    def pipe(k, _):
        i0 = k * NBUF
        for b in range(NBUF):
            wait(b)
            extract(i0 + b, b)
            fire(i0 + b + NBUF, b)
        return 0

    lax.fori_loop(0, B_PER_W // NBUF - 1, pipe, 0)
    for t in range(B_PER_W - NBUF, B_PER_W):
        b = t % NBUF
        wait(b)
        extract(t, b)

    pltpu.sync_copy(blk_v, outT3_hbm.at[:, :, pl.ds(wid * B_PER_W, B_PER_W)])


def kernel(labels, table):
    tableT3 = table.T.reshape(8, 8, V)
    labels_2d = labels.astype(jnp.int32).reshape(NW, B_PER_W)
    outT3 = _embed(tableT3, labels_2d)
    return outT3.reshape(D, B).T


# speedup vs baseline: 3.0136x; 1.0033x over previous
import functools

import jax
import jax.numpy as jnp
from jax import lax
from jax.experimental import pallas as pl
from jax.experimental.pallas import tpu as pltpu
from jax.experimental.pallas import tpu_sc as plsc

B = 16384
D = 64
V = 1000001
NC = 2
NS = 16
NW = NC * NS
B_PER_W = B // NW
L = 16
G = 128
NBUF = 8

_mesh = plsc.VectorSubcoreMesh(core_axis_name="c", subcore_axis_name="s")


@functools.partial(
    pl.kernel,
    mesh=_mesh,
    out_type=jax.ShapeDtypeStruct((8, 8, B), jnp.float32),
    scratch_types=[
        pltpu.VMEM((B_PER_W,), jnp.int32),
        pltpu.VMEM((NBUF, 8, 8, G), jnp.float32),
        pltpu.VMEM((NBUF, L), jnp.int32),
        pltpu.VMEM((8, 8, B_PER_W), jnp.float32),
    ]
    + [pltpu.SemaphoreType.DMA] * NBUF,
    compiler_params=pltpu.CompilerParams(
        use_tc_tiling_on_sc=True, needs_layout_passes=False
    ),
)
def _embed(tableT3_hbm, labels_hbm, outT3_hbm, lab_v, gran_v, cw_v, blk_v,
           *sems):
    wid = lax.axis_index("s") * NC + lax.axis_index("c")
    pltpu.sync_copy(labels_hbm.at[wid], lab_v)

    lane_iota = lax.iota(jnp.int32, L)
    ones = jnp.ones((L,), jnp.int32)
    zeros = jnp.zeros((L,), jnp.int32)
    eight = jnp.full((L,), 8, jnp.int32)
    pair_i0 = lax.div(lane_iota, eight)
    pair_a = lax.rem(lane_iota, eight)

    def fire(i, b):
        grp = lab_v[pl.ds((i // L) * L, L)]
        sel = jnp.where(lax.eq(lane_iota, lax.rem(i, L) * ones), grp, zeros)
        c = lax.reduce_max(sel, axes=(0,))
        base = pl.multiple_of((c // G) * G, G)
        cw_v[b, :] = lax.rem(c, G) * ones
        pltpu.async_copy(
            tableT3_hbm.at[:, :, pl.ds(base, G)], gran_v.at[b], sems[b]
        )

    def wait(b):
        pltpu.make_async_copy(
            tableT3_hbm.at[:, :, pl.ds(0, G)], gran_v.at[b], sems[b]
        ).wait()

    def extract(i, b):
        cw = cw_v[b, :]
        ivec = i * ones
        buf = gran_v.at[b]
        for g in range(4):
            i_idx = lax.add(pair_i0, jnp.full((L,), g * 2, jnp.int32))
            vals = plsc.load_gather(buf, [i_idx, pair_a, cw])
            plsc.store_scatter(blk_v, [i_idx, pair_a, ivec], vals)

    for b in range(NBUF):
        fire(b, b)

    def pipe(k, _):
        i0 = k * NBUF
        for b in range(NBUF):
            wait(b)
            extract(i0 + b, b)
            fire(i0 + b + NBUF, b)
        return 0

    lax.fori_loop(0, B_PER_W // NBUF - 1, pipe, 0)
    for t in range(B_PER_W - NBUF, B_PER_W):
        b = t % NBUF
        wait(b)
        extract(t, b)

    pltpu.sync_copy(blk_v, outT3_hbm.at[:, :, pl.ds(wid * B_PER_W, B_PER_W)])


def kernel(labels, table):
    tableT3 = table.T.reshape(8, 8, V)
    labels_2d = labels.astype(jnp.int32).reshape(NW, B_PER_W)
    outT3 = _embed(tableT3, labels_2d)
    return outT3.reshape(D, B).T


# speedup vs baseline: 3.0191x; 1.0018x over previous
import functools

import jax
import jax.numpy as jnp
from jax import lax
from jax.experimental import pallas as pl
from jax.experimental.pallas import tpu as pltpu
from jax.experimental.pallas import tpu_sc as plsc

B = 16384
D = 64
V = 1000001
NC = 2
NS = 16
NW = NC * NS
B_PER_W = B // NW
L = 16
G = 128
NBUF = 8

_mesh = plsc.VectorSubcoreMesh(core_axis_name="c", subcore_axis_name="s")


@functools.partial(
    pl.kernel,
    mesh=_mesh,
    out_type=jax.ShapeDtypeStruct((8, 8, B), jnp.float32),
    scratch_types=[
        pltpu.VMEM((B_PER_W,), jnp.int32),
        pltpu.VMEM((NBUF, 8, 8, G), jnp.float32),
        pltpu.VMEM((NBUF, L), jnp.int32),
        pltpu.VMEM((8, 8, B_PER_W), jnp.float32),
    ]
    + [pltpu.SemaphoreType.DMA] * NBUF,
    compiler_params=pltpu.CompilerParams(
        use_tc_tiling_on_sc=True, needs_layout_passes=False
    ),
)
def _embed(tableT3_hbm, labels_hbm, outT3_hbm, lab_v, gran_v, cw_v, blk_v,
           *sems):
    wid = lax.axis_index("s") * NC + lax.axis_index("c")
    pltpu.sync_copy(labels_hbm.at[pl.ds(wid * B_PER_W, B_PER_W)], lab_v)

    lane_iota = lax.iota(jnp.int32, L)
    ones = jnp.ones((L,), jnp.int32)
    zeros = jnp.zeros((L,), jnp.int32)
    eight = jnp.full((L,), 8, jnp.int32)
    pair_i0 = lax.div(lane_iota, eight)
    pair_a = lax.rem(lane_iota, eight)

    def fire(i, b):
        grp = lab_v[pl.ds((i // L) * L, L)]
        sel = jnp.where(lax.eq(lane_iota, lax.rem(i, L) * ones), grp, zeros)
        c = lax.reduce_max(sel, axes=(0,))
        base = pl.multiple_of((c // G) * G, G)
        cw_v[b, :] = lax.rem(c, G) * ones
        pltpu.async_copy(
            tableT3_hbm.at[:, :, pl.ds(base, G)], gran_v.at[b], sems[b]
        )

    def wait(b):
        pltpu.make_async_copy(
            tableT3_hbm.at[:, :, pl.ds(0, G)], gran_v.at[b], sems[b]
        ).wait()

    def extract(i, b):
        cw = cw_v[b, :]
        ivec = i * ones
        buf = gran_v.at[b]
        for g in range(4):
            i_idx = lax.add(pair_i0, jnp.full((L,), g * 2, jnp.int32))
            vals = plsc.load_gather(buf, [i_idx, pair_a, cw])
            plsc.store_scatter(blk_v, [i_idx, pair_a, ivec], vals)

    for b in range(NBUF):
        fire(b, b)

    def pipe(k, _):
        i0 = k * NBUF
        for b in range(NBUF):
            wait(b)
            extract(i0 + b, b)
            fire(i0 + b + NBUF, b)
        return 0

    lax.fori_loop(0, B_PER_W // NBUF - 1, pipe, 0)
    for t in range(B_PER_W - NBUF, B_PER_W):
        b = t % NBUF
        wait(b)
        extract(t, b)

    pltpu.sync_copy(blk_v, outT3_hbm.at[:, :, pl.ds(wid * B_PER_W, B_PER_W)])


def kernel(labels, table):
    tableT3 = table.T.reshape(8, 8, V)
    outT3 = _embed(tableT3, labels.astype(jnp.int32))
    return outT3.reshape(D, B).T


# speedup vs baseline: 3.0576x; 1.0128x over previous
import functools

import jax
import jax.numpy as jnp
from jax import lax
from jax.experimental import pallas as pl
from jax.experimental.pallas import tpu as pltpu
from jax.experimental.pallas import tpu_sc as plsc

B = 16384
D = 64
V = 1000001
NC = 2
NS = 16
NW = NC * NS
B_PER_W = B // NW
L = 16
G = 128
NBUF = 11

_mesh = plsc.VectorSubcoreMesh(core_axis_name="c", subcore_axis_name="s")


@functools.partial(
    pl.kernel,
    mesh=_mesh,
    out_type=jax.ShapeDtypeStruct((8, 8, B), jnp.float32),
    scratch_types=[
        pltpu.VMEM((B_PER_W,), jnp.int32),
        pltpu.VMEM((NBUF, 8, 8, G), jnp.float32),
        pltpu.VMEM((NBUF, L), jnp.int32),
        pltpu.VMEM((8, 8, B_PER_W), jnp.float32),
    ]
    + [pltpu.SemaphoreType.DMA] * NBUF,
    compiler_params=pltpu.CompilerParams(
        use_tc_tiling_on_sc=True, needs_layout_passes=False
    ),
)
def _embed(tableT3_hbm, labels_hbm, outT3_hbm, lab_v, gran_v, cw_v, blk_v,
           *sems):
    wid = lax.axis_index("s") * NC + lax.axis_index("c")
    pltpu.sync_copy(labels_hbm.at[pl.ds(wid * B_PER_W, B_PER_W)], lab_v)

    lane_iota = lax.iota(jnp.int32, L)
    ones = jnp.ones((L,), jnp.int32)
    zeros = jnp.zeros((L,), jnp.int32)
    eight = jnp.full((L,), 8, jnp.int32)
    pair_i0 = lax.div(lane_iota, eight)
    pair_a = lax.rem(lane_iota, eight)

    def fire(i, b):
        grp = lab_v[pl.ds((i // L) * L, L)]
        sel = jnp.where(lax.eq(lane_iota, lax.rem(i, L) * ones), grp, zeros)
        c = lax.reduce_max(sel, axes=(0,))
        base = pl.multiple_of((c // G) * G, G)
        cw_v[b, :] = lax.rem(c, G) * ones
        pltpu.async_copy(
            tableT3_hbm.at[:, :, pl.ds(base, G)], gran_v.at[b], sems[b]
        )

    def wait(b):
        pltpu.make_async_copy(
            tableT3_hbm.at[:, :, pl.ds(0, G)], gran_v.at[b], sems[b]
        ).wait()

    def extract(i, b):
        cw = cw_v[b, :]
        ivec = i * ones
        buf = gran_v.at[b]
        for g in range(4):
            i_idx = lax.add(pair_i0, jnp.full((L,), g * 2, jnp.int32))
            vals = plsc.load_gather(buf, [i_idx, pair_a, cw])
            plsc.store_scatter(blk_v, [i_idx, pair_a, ivec], vals)

    for b in range(NBUF):
        fire(b, b)

    def pipe(k, _):
        i0 = k * NBUF
        for b in range(NBUF):
            wait(b)
            extract(i0 + b, b)
            fire(i0 + b + NBUF, b)
        return 0

    NFULL = (B_PER_W - NBUF) // NBUF
    REM = B_PER_W - (NFULL + 1) * NBUF
    lax.fori_loop(0, NFULL, pipe, 0)
    for b in range(NBUF):
        wait(b)
        extract(NFULL * NBUF + b, b)
        if b < REM:
            fire((NFULL + 1) * NBUF + b, b)
    for b in range(REM):
        wait(b)
        extract((NFULL + 1) * NBUF + b, b)

    pltpu.sync_copy(blk_v, outT3_hbm.at[:, :, pl.ds(wid * B_PER_W, B_PER_W)])


def kernel(labels, table):
    tableT3 = table.T.reshape(8, 8, V)
    outT3 = _embed(tableT3, labels.astype(jnp.int32))
    return outT3.reshape(D, B).T
